# speedup vs baseline: 3.3366x; 3.3366x over previous
import functools

import jax
import jax.numpy as jnp
from jax import lax
from jax.experimental import pallas as pl
from jax.experimental.pallas import tpu as pltpu

B, T, N = 32, 12, 1024
INPUT_DIM = 3
IN_EMB, TOD_EMB, DOW_EMB, ADP_EMB = 24, 24, 24, 80
STEPS_PER_DAY = 288
DOW_PAD = 8
OUT_DIM = IN_EMB + TOD_EMB + DOW_EMB + ADP_EMB


def _body(x_ref, w_ref, b_ref, tod_ref, dow_ref, adp_ref, out_ref):
    xb = x_ref[0, 0]
    xi = xb[:, :INPUT_DIM]
    h = jnp.dot(xi, w_ref[...], preferred_element_type=jnp.float32) + b_ref[0]

    tod_idx = (xb[:, INPUT_DIM] * STEPS_PER_DAY).astype(jnp.int32)
    tod_idx = jnp.clip(tod_idx, 0, STEPS_PER_DAY - 1)
    oh_tod = (lax.broadcasted_iota(jnp.int32, (N, STEPS_PER_DAY), 1)
              == tod_idx[:, None]).astype(jnp.float32)
    tod_emb = jnp.dot(oh_tod, tod_ref[...], preferred_element_type=jnp.float32)

    dow_idx = xb[:, INPUT_DIM + 1].astype(jnp.int32)
    dow_idx = jnp.clip(dow_idx, 0, 6)
    oh_dow = (lax.broadcasted_iota(jnp.int32, (N, DOW_PAD), 1)
              == dow_idx[:, None]).astype(jnp.float32)
    dow_emb = jnp.dot(oh_dow, dow_ref[...], preferred_element_type=jnp.float32)

    out_ref[0, 0, :, 0:IN_EMB] = h
    out_ref[0, 0, :, IN_EMB:IN_EMB + TOD_EMB] = tod_emb
    out_ref[0, 0, :, IN_EMB + TOD_EMB:IN_EMB + TOD_EMB + DOW_EMB] = dow_emb
    out_ref[0, 0, :, IN_EMB + TOD_EMB + DOW_EMB:OUT_DIM] = adp_ref[0]


@functools.partial(jax.jit, static_argnames=("interpret",))
def kernel(x, W_proj, b_proj, tod_table, dow_table, adaptive, interpret=False):
    dow_pad = jnp.zeros((DOW_PAD, DOW_EMB), jnp.float32).at[:7].set(dow_table)
    grid = (T, B)
    return pl.pallas_call(
        _body,
        grid=grid,
        in_specs=[
            pl.BlockSpec((1, 1, N, INPUT_DIM + 2), lambda t, b: (b, t, 0, 0)),
            pl.BlockSpec((INPUT_DIM, IN_EMB), lambda t, b: (0, 0)),
            pl.BlockSpec((1, IN_EMB), lambda t, b: (0, 0)),
            pl.BlockSpec((STEPS_PER_DAY, TOD_EMB), lambda t, b: (0, 0)),
            pl.BlockSpec((DOW_PAD, DOW_EMB), lambda t, b: (0, 0)),
            pl.BlockSpec((1, N, ADP_EMB), lambda t, b: (t, 0, 0)),
        ],
        out_specs=pl.BlockSpec((1, 1, N, OUT_DIM), lambda t, b: (b, t, 0, 0)),
        out_shape=jax.ShapeDtypeStruct((B, T, N, OUT_DIM), jnp.float32),
        compiler_params=pltpu.CompilerParams(
            dimension_semantics=("arbitrary", "arbitrary"),
        ),
        interpret=interpret,
    )(x, W_proj, b_proj.reshape(1, IN_EMB), tod_table, dow_pad, adaptive)


# speedup vs baseline: 3.9146x; 1.1732x over previous
import functools

import jax
import jax.numpy as jnp
from jax import lax
from jax.experimental import pallas as pl
from jax.experimental.pallas import tpu as pltpu

B, T, N = 32, 12, 1024
INPUT_DIM = 3
IN_EMB, TOD_EMB, DOW_EMB, ADP_EMB = 24, 24, 24, 80
STEPS_PER_DAY = 288
DOW_PAD = 8
OUT_DIM = IN_EMB + TOD_EMB + DOW_EMB + ADP_EMB


BB = 4


def _body(x_ref, w_ref, b_ref, tod_ref, dow_ref, adp_ref, out_ref):
    M = BB * N
    xb = x_ref[:, 0].reshape(M, INPUT_DIM + 2)
    xi = xb[:, :INPUT_DIM]
    h = jnp.dot(xi, w_ref[...], preferred_element_type=jnp.float32) + b_ref[0]

    tod_idx = (xb[:, INPUT_DIM] * STEPS_PER_DAY).astype(jnp.int32)
    tod_idx = jnp.clip(tod_idx, 0, STEPS_PER_DAY - 1)
    oh_tod = (lax.broadcasted_iota(jnp.int32, (M, STEPS_PER_DAY), 1)
              == tod_idx[:, None]).astype(jnp.float32)
    tod_emb = jnp.dot(oh_tod, tod_ref[...], preferred_element_type=jnp.float32)

    dow_idx = xb[:, INPUT_DIM + 1].astype(jnp.int32)
    dow_idx = jnp.clip(dow_idx, 0, 6)
    oh_dow = (lax.broadcasted_iota(jnp.int32, (M, DOW_PAD), 1)
              == dow_idx[:, None]).astype(jnp.float32)
    dow_emb = jnp.dot(oh_dow, dow_ref[...], preferred_element_type=jnp.float32)

    h4 = h.reshape(BB, N, IN_EMB)
    t4 = tod_emb.reshape(BB, N, TOD_EMB)
    d4 = dow_emb.reshape(BB, N, DOW_EMB)
    for i in range(BB):
        out_ref[i, 0, :, 0:IN_EMB] = h4[i]
        out_ref[i, 0, :, IN_EMB:IN_EMB + TOD_EMB] = t4[i]
        out_ref[i, 0, :, IN_EMB + TOD_EMB:IN_EMB + TOD_EMB + DOW_EMB] = d4[i]
        out_ref[i, 0, :, IN_EMB + TOD_EMB + DOW_EMB:OUT_DIM] = adp_ref[0]


@functools.partial(jax.jit, static_argnames=("interpret",))
def kernel(x, W_proj, b_proj, tod_table, dow_table, adaptive, interpret=False):
    dow_pad = jnp.zeros((DOW_PAD, DOW_EMB), jnp.float32).at[:7].set(dow_table)
    grid = (T, B // BB)
    return pl.pallas_call(
        _body,
        grid=grid,
        in_specs=[
            pl.BlockSpec((BB, 1, N, INPUT_DIM + 2), lambda t, b: (b, t, 0, 0)),
            pl.BlockSpec((INPUT_DIM, IN_EMB), lambda t, b: (0, 0)),
            pl.BlockSpec((1, IN_EMB), lambda t, b: (0, 0)),
            pl.BlockSpec((STEPS_PER_DAY, TOD_EMB), lambda t, b: (0, 0)),
            pl.BlockSpec((DOW_PAD, DOW_EMB), lambda t, b: (0, 0)),
            pl.BlockSpec((1, N, ADP_EMB), lambda t, b: (t, 0, 0)),
        ],
        out_specs=pl.BlockSpec((BB, 1, N, OUT_DIM), lambda t, b: (b, t, 0, 0)),
        out_shape=jax.ShapeDtypeStruct((B, T, N, OUT_DIM), jnp.float32),
        compiler_params=pltpu.CompilerParams(
            dimension_semantics=("arbitrary", "arbitrary"),
        ),
        interpret=interpret,
    )(x, W_proj, b_proj.reshape(1, IN_EMB), tod_table, dow_pad, adaptive)


# speedup vs baseline: 4.1971x; 1.0722x over previous
import functools

import jax
import jax.numpy as jnp
from jax import lax
from jax.experimental import pallas as pl
from jax.experimental.pallas import tpu as pltpu

B, T, N = 32, 12, 1024
INPUT_DIM = 3
IN_EMB, TOD_EMB, DOW_EMB, ADP_EMB = 24, 24, 24, 80
STEPS_PER_DAY = 288
OUT_DIM = IN_EMB + TOD_EMB + DOW_EMB + ADP_EMB
K_OH = 304
BB = 4


def _body(x_ref, w_ref, b_ref, tcat_ref, adp_ref, out_ref):
    M = BB * N
    xb = x_ref[:, 0].reshape(M, INPUT_DIM + 2)
    xi = xb[:, :INPUT_DIM]
    h = jnp.dot(xi, w_ref[...], preferred_element_type=jnp.float32) + b_ref[0]

    tod_idx = (xb[:, INPUT_DIM] * STEPS_PER_DAY).astype(jnp.int32)
    tod_idx = jnp.clip(tod_idx, 0, STEPS_PER_DAY - 1)
    dow_idx = xb[:, INPUT_DIM + 1].astype(jnp.int32)
    dow_idx = jnp.clip(dow_idx, 0, 6)

    lane = lax.broadcasted_iota(jnp.int32, (M, K_OH), 1)
    oh = ((lane == tod_idx[:, None])
          | (lane == dow_idx[:, None] + STEPS_PER_DAY)).astype(jnp.bfloat16)
    emb = jnp.dot(oh, tcat_ref[...], preferred_element_type=jnp.float32)

    h4 = h.reshape(BB, N, IN_EMB)
    e4 = emb.reshape(BB, N, OUT_DIM)
    for i in range(BB):
        out_ref[i, 0] = e4[i]
        out_ref[i, 0, :, 0:IN_EMB] = h4[i]
        out_ref[i, 0, :, IN_EMB + TOD_EMB + DOW_EMB:OUT_DIM] = adp_ref[0]


@functools.partial(jax.jit, static_argnames=("interpret",))
def kernel(x, W_proj, b_proj, tod_table, dow_table, adaptive, interpret=False):
    tcat = jnp.zeros((K_OH, OUT_DIM), jnp.bfloat16)
    tcat = tcat.at[:STEPS_PER_DAY, IN_EMB:IN_EMB + TOD_EMB].set(
        tod_table.astype(jnp.bfloat16))
    tcat = tcat.at[STEPS_PER_DAY:STEPS_PER_DAY + 7,
                   IN_EMB + TOD_EMB:IN_EMB + TOD_EMB + DOW_EMB].set(
        dow_table.astype(jnp.bfloat16))
    grid = (T, B // BB)
    return pl.pallas_call(
        _body,
        grid=grid,
        in_specs=[
            pl.BlockSpec((BB, 1, N, INPUT_DIM + 2), lambda t, b: (b, t, 0, 0)),
            pl.BlockSpec((INPUT_DIM, IN_EMB), lambda t, b: (0, 0)),
            pl.BlockSpec((1, IN_EMB), lambda t, b: (0, 0)),
            pl.BlockSpec((K_OH, OUT_DIM), lambda t, b: (0, 0)),
            pl.BlockSpec((1, N, ADP_EMB), lambda t, b: (t, 0, 0)),
        ],
        out_specs=pl.BlockSpec((BB, 1, N, OUT_DIM), lambda t, b: (b, t, 0, 0)),
        out_shape=jax.ShapeDtypeStruct((B, T, N, OUT_DIM), jnp.float32),
        compiler_params=pltpu.CompilerParams(
            dimension_semantics=("arbitrary", "arbitrary"),
        ),
        interpret=interpret,
    )(x, W_proj, b_proj.reshape(1, IN_EMB), tcat, adaptive)
